# baseline (speedup 1.0000x reference)
import functools

import jax
import jax.numpy as jnp
from jax import lax
from jax.experimental import pallas as pl
from jax.experimental.pallas import tpu as pltpu
from jax.experimental.pallas import tpu_sc as plsc

_K = 64
_L = 16
_NBINS = 256
_ROWS = 128
_N = 32768
_NCHUNK = _N // _L

_SIGN = -2147483648
_M31 = 0x7FFFFFFF


def _i32(c):
    return jnp.asarray(c, jnp.int32)


def _f32_key(v):
    b = plsc.bitcast(v, jnp.int32)
    return jnp.where(b < 0, b ^ _i32(_M31), b)


def _sc_body(x_hbm, o_hbm, xv, hist, folded):
    nc = 2
    wid = lax.axis_index("s") * nc + lax.axis_index("c")
    lane = lax.iota(jnp.int32, _L)
    lanebase = lane * _NBINS
    ones = jnp.ones((_L,), jnp.int32)
    zeros16 = jnp.zeros((_L,), jnp.int32)
    rev_lane = jnp.int32(_L - 1) - lane

    def do_row(j, _):
        row = wid * (_ROWS // 32) + j
        pltpu.sync_copy(x_hbm.at[row], xv)

        def round_fn(rnd_shift, prefix, need):
            def zero_hist(i, _):
                hist[pl.ds(i * _L, _L)] = zeros16
                return 0
            lax.fori_loop(0, (_NBINS * _L) // _L, zero_hist, 0)

            match_shift = rnd_shift + 8

            def hpass(i, _):
                v = xv[pl.ds(i * _L, _L)]
                ukey = _f32_key(v) ^ _SIGN
                dig = lax.shift_right_logical(ukey, rnd_shift) & jnp.int32(0xFF)
                if rnd_shift == 24:
                    plsc.addupdate_scatter(hist, [dig + lanebase], ones)
                else:
                    pref = lax.shift_right_logical(ukey, match_shift)
                    m = pref == prefix
                    plsc.addupdate_scatter(hist, [dig + lanebase], ones,
                                           mask=m)
                return 0

            lax.fori_loop(0, _NCHUNK, hpass, 0)

            def fold(c, _):
                acc = hist[pl.ds(c * _L, _L)]
                for l in range(1, _L):
                    acc = acc + hist[pl.ds(l * _NBINS + c * _L, _L)]
                folded[pl.ds(c * _L, _L)] = acc
                return 0

            lax.fori_loop(0, _NBINS // _L, fold, 0)

            def scan(i, carry):
                run, dstar = carry
                cc = (_NBINS // _L - 1) - i
                v = folded[pl.ds(cc * _L, _L)]
                rv = lax.rev(v, dimensions=(0,))
                sfx = plsc.cumsum(rv) + run
                digs_desc = jnp.int32(cc * _L) + rev_lane
                cand = jnp.where(sfx >= need, digs_desc, jnp.int32(-1))
                dstar = jnp.maximum(dstar, jnp.max(cand))
                return jnp.max(sfx), dstar

            _, dstar = lax.fori_loop(
                0, _NBINS // _L, scan, (jnp.int32(0), jnp.int32(-1)))

            def counts(c, carry):
                cgt, sge = carry
                digs = jnp.int32(c * _L) + lane
                v = folded[pl.ds(c * _L, _L)]
                cgt = cgt + jnp.sum(jnp.where(digs > dstar, v, 0))
                sge = sge + jnp.sum(jnp.where(digs >= dstar, v, 0))
                return cgt, sge

            cgt, sge = lax.fori_loop(
                0, _NBINS // _L, counts, (jnp.int32(0), jnp.int32(0)))
            return dstar, cgt, sge

        need = jnp.int32(_K)
        prefix = jnp.int32(0)
        total_eq = jnp.int32(0)
        for rnd, shift in enumerate((24, 16, 8, 0)):
            dstar, cgt, sge = round_fn(shift, prefix, need)
            prefix = prefix * jnp.int32(_NBINS) + dstar
            need = need - cgt
            if shift == 0:
                total_eq = sge - cgt

        t_u = prefix
        t_s = t_u ^ _SIGN
        r = need

        def simple_pass(_):
            def body(i, _):
                v = xv[pl.ds(i * _L, _L)]
                key = _f32_key(v)
                xv[pl.ds(i * _L, _L)] = jnp.where(
                    key >= t_s, v, jnp.float32(0.0))
                return 0
            lax.fori_loop(0, _NCHUNK, body, 0)
            return 0

        def tie_pass(_):
            def body(i, carry):
                v = xv[pl.ds(i * _L, _L)]
                key = _f32_key(v)
                gt = key > t_s
                eq = key == t_s
                pc = plsc.cumsum(eq.astype(jnp.int32)) + carry
                keep = gt | (eq & (pc <= r))
                xv[pl.ds(i * _L, _L)] = jnp.where(keep, v, jnp.float32(0.0))
                return jnp.max(pc)
            lax.fori_loop(0, _NCHUNK, body, jnp.int32(0))
            return 0

        lax.cond(total_eq == r, simple_pass, tie_pass, 0)

        pltpu.sync_copy(xv, o_hbm.at[row])
        return 0

    lax.fori_loop(0, _ROWS // 32, do_row, 0)


def kernel(x):
    mesh = plsc.VectorSubcoreMesh(core_axis_name="c", subcore_axis_name="s")
    f = functools.partial(
        pl.kernel,
        out_type=jax.ShapeDtypeStruct((_ROWS, _N), jnp.float32),
        mesh=mesh,
        compiler_params=pltpu.CompilerParams(needs_layout_passes=False),
        scratch_types=[
            pltpu.VMEM((_N,), jnp.float32),
            pltpu.VMEM((_NBINS * _L,), jnp.int32),
            pltpu.VMEM((_NBINS,), jnp.int32),
        ],
    )(_sc_body)
    return f(x)


# speedup vs baseline: 1.6391x; 1.6391x over previous
import functools

import jax
import jax.numpy as jnp
from jax import lax
from jax.experimental import pallas as pl
from jax.experimental.pallas import tpu as pltpu
from jax.experimental.pallas import tpu_sc as plsc

_K = 64
_L = 16
_NBINS = 256
_ROWS = 128
_N = 32768
_NCHUNK = _N // _L
_UNROLL = 8

_SIGN = -2147483648
_M31 = 0x7FFFFFFF


def _ukey(v, c31):
    b = plsc.bitcast(v, jnp.int32)
    return b ^ (lax.shift_right_arithmetic(b, c31) | jnp.int32(_SIGN))


def _key_to_f32(t_s16):
    bits = jnp.where(t_s16 < 0, t_s16 ^ jnp.int32(_M31), t_s16)
    return plsc.bitcast(bits, jnp.float32)


def _sc_body(x_hbm, o_hbm, xv, hist):
    nc = 2
    wid = lax.axis_index("s") * nc + lax.axis_index("c")
    lane = lax.iota(jnp.int32, _L)
    c31 = jnp.full((_L,), 31, jnp.int32)
    ones = jnp.ones((_L,), jnp.int32)
    zeros16i = jnp.zeros((_L,), jnp.int32)
    zeros16f = jnp.zeros((_L,), jnp.float32)
    rev_lane = jnp.int32(_L - 1) - lane
    shift_vecs = {s: jnp.full((_L,), s, jnp.int32) for s in (8, 16, 24)}

    def run_round(shift, prefix, need):
        for z in range(_NBINS // _L):
            hist[pl.ds(z * _L, _L)] = zeros16i

        if shift == 24:
            def hpass(i, _):
                base = i * (_L * _UNROLL)
                for k in range(_UNROLL):
                    v = xv[pl.ds(base + k * _L, _L)]
                    u = _ukey(v, c31)
                    dig = lax.shift_right_logical(u, shift_vecs[24])
                    plsc.addupdate_scatter(hist, [dig], ones)
                return 0
        else:
            msk_shift = shift_vecs[shift + 8]
            def hpass(i, _):
                base = i * (_L * _UNROLL)
                for k in range(_UNROLL):
                    v = xv[pl.ds(base + k * _L, _L)]
                    u = _ukey(v, c31)
                    m = lax.shift_right_logical(u, msk_shift) == prefix
                    if shift == 0:
                        dig = u & jnp.int32(0xFF)
                    else:
                        dig = (lax.shift_right_logical(u, shift_vecs[shift])
                               & jnp.int32(0xFF))
                    plsc.addupdate_scatter(hist, [dig], ones, mask=m)
                return 0

        lax.fori_loop(0, _NCHUNK // _UNROLL, hpass, 0, unroll=False)

        def scan(i, carry):
            run, dstar = carry
            cc = (_NBINS // _L - 1) - i
            vv = hist[pl.ds(cc * _L, _L)]
            rv = lax.rev(vv, dimensions=(0,))
            sfx = plsc.cumsum(rv) + run
            digs_desc = jnp.int32(cc * _L) + rev_lane
            cand = jnp.where(sfx >= need, digs_desc, jnp.int32(-1))
            return jnp.max(sfx), jnp.maximum(dstar, jnp.max(cand))

        _, dstar = lax.fori_loop(0, _NBINS // _L, scan,
                                 (jnp.int32(0), jnp.int32(-1)))

        def counts(c, carry):
            cgt, sge = carry
            digs = jnp.int32(c * _L) + lane
            vv = hist[pl.ds(c * _L, _L)]
            cgt = cgt + jnp.sum(jnp.where(digs > dstar, vv, 0))
            sge = sge + jnp.sum(jnp.where(digs >= dstar, vv, 0))
            return cgt, sge

        cgt, sge = lax.fori_loop(0, _NBINS // _L, counts,
                                 (jnp.int32(0), jnp.int32(0)))
        return dstar, cgt, sge

    def resolve(shift, prefix, need):
        dstar, cgt, sge = run_round(shift, prefix, need)
        prefix2 = prefix * jnp.int32(_NBINS) + dstar
        need2 = need - cgt
        m = sge - cgt
        if shift == 0:
            return prefix2, need2, m

        def exit_fn(op):
            p2, n2 = op
            t_u = p2 * jnp.int32(1 << shift)
            return t_u, n2, n2

        def cont_fn(op):
            p2, n2 = op
            return resolve(shift - 8, p2, n2)

        return lax.cond(m == need2, exit_fn, cont_fn, (prefix2, need2))

    def do_row(j, _):
        row = wid * (_ROWS // 32) + j
        pltpu.sync_copy(x_hbm.at[row], xv)

        t_u, r, total_eq = resolve(24, jnp.int32(0), jnp.int32(_K))
        t_s = t_u ^ jnp.int32(_SIGN)
        tf = _key_to_f32(jnp.full((_L,), 0, jnp.int32) + t_s)

        def simple_pass(_o):
            def body(i, _):
                base = i * (_L * _UNROLL)
                for k in range(_UNROLL):
                    v = xv[pl.ds(base + k * _L, _L)]
                    xv[pl.ds(base + k * _L, _L)] = jnp.where(
                        v >= tf, v, zeros16f)
                return 0
            lax.fori_loop(0, _NCHUNK // _UNROLL, body, 0, unroll=False)
            return 0

        def tie_pass(_o):
            def body(i, carry):
                v = xv[pl.ds(i * _L, _L)]
                gt = v > tf
                eq = v == tf
                pc = plsc.cumsum(eq.astype(jnp.int32)) + carry
                keep = gt | (eq & (pc <= r))
                xv[pl.ds(i * _L, _L)] = jnp.where(keep, v, zeros16f)
                return jnp.max(pc)
            lax.fori_loop(0, _NCHUNK, body, jnp.int32(0))
            return 0

        lax.cond(total_eq == r, simple_pass, tie_pass, 0)

        pltpu.sync_copy(xv, o_hbm.at[row])
        return 0

    lax.fori_loop(0, _ROWS // 32, do_row, 0)


def kernel(x):
    mesh = plsc.VectorSubcoreMesh(core_axis_name="c", subcore_axis_name="s")
    f = functools.partial(
        pl.kernel,
        out_type=jax.ShapeDtypeStruct((_ROWS, _N), jnp.float32),
        mesh=mesh,
        compiler_params=pltpu.CompilerParams(needs_layout_passes=False),
        scratch_types=[
            pltpu.VMEM((_N,), jnp.float32),
            pltpu.VMEM((_NBINS,), jnp.int32),
        ],
    )(_sc_body)
    return f(x)


# speedup vs baseline: 4.8334x; 2.9488x over previous
import functools

import jax
import jax.numpy as jnp
from jax import lax
from jax.experimental import pallas as pl
from jax.experimental.pallas import tpu as pltpu
from jax.experimental.pallas import tpu_sc as plsc

_K = 64
_L = 16
_NBINS = 256
_ROWS = 128
_N = 32768
_NCHUNK = _N // _L
_UNROLL = 8

_SIGN = -2147483648
_M31 = 0x7FFFFFFF


def _ukey(v, c31):
    b = plsc.bitcast(v, jnp.int32)
    return b ^ (lax.shift_right_arithmetic(b, c31) | jnp.int32(_SIGN))


def _key_to_f32(t_s16):
    bits = jnp.where(t_s16 < 0, t_s16 ^ jnp.int32(_M31), t_s16)
    return plsc.bitcast(bits, jnp.float32)


def _sc_body(x_hbm, o_hbm, xv, hist):
    nc = 2
    wid = lax.axis_index("s") * nc + lax.axis_index("c")
    lane = lax.iota(jnp.int32, _L)
    c31 = jnp.full((_L,), 31, jnp.int32)
    ones = jnp.ones((_L,), jnp.int32)
    zeros16i = jnp.zeros((_L,), jnp.int32)
    zeros16f = jnp.zeros((_L,), jnp.float32)
    rev_lane = jnp.int32(_L - 1) - lane
    shift_vecs = {s: jnp.full((_L,), s, jnp.int32) for s in (8, 16, 24)}

    def run_round(shift, prefix, need):
        for z in range(_NBINS // _L):
            hist[pl.ds(z * _L, _L)] = zeros16i

        if shift == 24:
            @plsc.parallel_loop(0, _NCHUNK, 1, unroll=_UNROLL)
            def _hpass(i):
                v = xv[pl.ds(i * _L, _L)]
                u = _ukey(v, c31)
                dig = lax.shift_right_logical(u, shift_vecs[24])
                plsc.addupdate_scatter(hist, [dig], ones)
        else:
            msk_shift = shift_vecs[shift + 8]

            @plsc.parallel_loop(0, _NCHUNK, 1, unroll=_UNROLL)
            def _hpass(i):
                v = xv[pl.ds(i * _L, _L)]
                u = _ukey(v, c31)
                m = lax.shift_right_logical(u, msk_shift) == prefix
                if shift == 0:
                    dig = u & jnp.int32(0xFF)
                else:
                    dig = (lax.shift_right_logical(u, shift_vecs[shift])
                           & jnp.int32(0xFF))
                plsc.addupdate_scatter(hist, [dig], ones, mask=m)

        def scan(i, carry):
            run, dstar = carry
            cc = (_NBINS // _L - 1) - i
            vv = hist[pl.ds(cc * _L, _L)]
            rv = lax.rev(vv, dimensions=(0,))
            sfx = plsc.cumsum(rv) + run
            digs_desc = jnp.int32(cc * _L) + rev_lane
            cand = jnp.where(sfx >= need, digs_desc, jnp.int32(-1))
            return jnp.max(sfx), jnp.maximum(dstar, jnp.max(cand))

        _, dstar = lax.fori_loop(0, _NBINS // _L, scan,
                                 (jnp.int32(0), jnp.int32(-1)))

        def counts(c, carry):
            cgt, sge = carry
            digs = jnp.int32(c * _L) + lane
            vv = hist[pl.ds(c * _L, _L)]
            cgt = cgt + jnp.sum(jnp.where(digs > dstar, vv, 0))
            sge = sge + jnp.sum(jnp.where(digs >= dstar, vv, 0))
            return cgt, sge

        cgt, sge = lax.fori_loop(0, _NBINS // _L, counts,
                                 (jnp.int32(0), jnp.int32(0)))
        return dstar, cgt, sge

    def resolve(shift, prefix, need):
        dstar, cgt, sge = run_round(shift, prefix, need)
        prefix2 = prefix * jnp.int32(_NBINS) + dstar
        need2 = need - cgt
        m = sge - cgt
        if shift == 0:
            return prefix2, need2, m

        def exit_fn(op):
            p2, n2 = op
            t_u = p2 * jnp.int32(1 << shift)
            return t_u, n2, n2

        def cont_fn(op):
            p2, n2 = op
            return resolve(shift - 8, p2, n2)

        return lax.cond(m == need2, exit_fn, cont_fn, (prefix2, need2))

    def do_row(j, _):
        row = wid * (_ROWS // 32) + j
        pltpu.sync_copy(x_hbm.at[row], xv)

        t_u, r, total_eq = resolve(24, jnp.int32(0), jnp.int32(_K))
        t_s = t_u ^ jnp.int32(_SIGN)
        tf = _key_to_f32(jnp.full((_L,), 0, jnp.int32) + t_s)

        def simple_pass(_o):
            @plsc.parallel_loop(0, _NCHUNK, 1, unroll=_UNROLL)
            def _body(i):
                v = xv[pl.ds(i * _L, _L)]
                xv[pl.ds(i * _L, _L)] = jnp.where(v >= tf, v, zeros16f)
            return 0

        def tie_pass(_o):
            def body(i, carry):
                v = xv[pl.ds(i * _L, _L)]
                gt = v > tf
                eq = v == tf
                pc = plsc.cumsum(eq.astype(jnp.int32)) + carry
                keep = gt | (eq & (pc <= r))
                xv[pl.ds(i * _L, _L)] = jnp.where(keep, v, zeros16f)
                return jnp.max(pc)
            lax.fori_loop(0, _NCHUNK, body, jnp.int32(0))
            return 0

        lax.cond(total_eq == r, simple_pass, tie_pass, 0)

        pltpu.sync_copy(xv, o_hbm.at[row])
        return 0

    lax.fori_loop(0, _ROWS // 32, do_row, 0)


def kernel(x):
    mesh = plsc.VectorSubcoreMesh(core_axis_name="c", subcore_axis_name="s")
    f = functools.partial(
        pl.kernel,
        out_type=jax.ShapeDtypeStruct((_ROWS, _N), jnp.float32),
        mesh=mesh,
        compiler_params=pltpu.CompilerParams(needs_layout_passes=False),
        scratch_types=[
            pltpu.VMEM((_N,), jnp.float32),
            pltpu.VMEM((_NBINS,), jnp.int32),
        ],
    )(_sc_body)
    return f(x)


# speedup vs baseline: 4.9559x; 1.0253x over previous
import functools

import jax
import jax.numpy as jnp
from jax import lax
from jax.experimental import pallas as pl
from jax.experimental.pallas import tpu as pltpu
from jax.experimental.pallas import tpu_sc as plsc

_K = 64
_L = 16
_NBINS = 256
_ROWS = 128
_N = 32768
_NCHUNK = _N // _L
_UNROLL = 8

_SIGN = -2147483648
_M31 = 0x7FFFFFFF


def _ukey(v, c31):
    b = plsc.bitcast(v, jnp.int32)
    return b ^ (lax.shift_right_arithmetic(b, c31) | jnp.int32(_SIGN))


def _key_to_f32(t_s16):
    bits = jnp.where(t_s16 < 0, t_s16 ^ jnp.int32(_M31), t_s16)
    return plsc.bitcast(bits, jnp.float32)


def _sc_body(x_hbm, o_hbm, xv0, xv1, xv2, hist, sem0, sem1, sem2):
    nc = 2
    wid = lax.axis_index("s") * nc + lax.axis_index("c")
    bufs = (xv0, xv1, xv2)
    sems = (sem0, sem1, sem2)
    lane = lax.iota(jnp.int32, _L)
    c31 = jnp.full((_L,), 31, jnp.int32)
    ones = jnp.ones((_L,), jnp.int32)
    zeros16i = jnp.zeros((_L,), jnp.int32)
    zeros16f = jnp.zeros((_L,), jnp.float32)
    rev_lane = jnp.int32(_L - 1) - lane
    shift_vecs = {s: jnp.full((_L,), s, jnp.int32) for s in (8, 16, 24)}

    def run_round(xv, shift, prefix, need):
        for z in range(_NBINS // _L):
            hist[pl.ds(z * _L, _L)] = zeros16i

        if shift == 24:
            @plsc.parallel_loop(0, _NCHUNK, 1, unroll=_UNROLL)
            def _hpass(i):
                v = xv[pl.ds(i * _L, _L)]
                u = _ukey(v, c31)
                dig = lax.shift_right_logical(u, shift_vecs[24])
                plsc.addupdate_scatter(hist, [dig], ones)
        else:
            msk_shift = shift_vecs[shift + 8]

            @plsc.parallel_loop(0, _NCHUNK, 1, unroll=_UNROLL)
            def _hpass(i):
                v = xv[pl.ds(i * _L, _L)]
                u = _ukey(v, c31)
                m = lax.shift_right_logical(u, msk_shift) == prefix
                if shift == 0:
                    dig = u & jnp.int32(0xFF)
                else:
                    dig = (lax.shift_right_logical(u, shift_vecs[shift])
                           & jnp.int32(0xFF))
                plsc.addupdate_scatter(hist, [dig], ones, mask=m)

        def scan(i, carry):
            run, dstar = carry
            cc = (_NBINS // _L - 1) - i
            vv = hist[pl.ds(cc * _L, _L)]
            rv = lax.rev(vv, dimensions=(0,))
            sfx = plsc.cumsum(rv) + run
            digs_desc = jnp.int32(cc * _L) + rev_lane
            cand = jnp.where(sfx >= need, digs_desc, jnp.int32(-1))
            return jnp.max(sfx), jnp.maximum(dstar, jnp.max(cand))

        _, dstar = lax.fori_loop(0, _NBINS // _L, scan,
                                 (jnp.int32(0), jnp.int32(-1)))

        def counts(c, carry):
            cgt, sge = carry
            digs = jnp.int32(c * _L) + lane
            vv = hist[pl.ds(c * _L, _L)]
            cgt = cgt + jnp.sum(jnp.where(digs > dstar, vv, 0))
            sge = sge + jnp.sum(jnp.where(digs >= dstar, vv, 0))
            return cgt, sge

        cgt, sge = lax.fori_loop(0, _NBINS // _L, counts,
                                 (jnp.int32(0), jnp.int32(0)))
        return dstar, cgt, sge

    def resolve(xv, shift, prefix, need):
        dstar, cgt, sge = run_round(xv, shift, prefix, need)
        prefix2 = prefix * jnp.int32(_NBINS) + dstar
        need2 = need - cgt
        m = sge - cgt
        if shift == 0:
            return prefix2, need2, m

        def exit_fn(op):
            p2, n2 = op
            t_u = p2 * jnp.int32(1 << shift)
            return t_u, n2, n2

        def cont_fn(op):
            p2, n2 = op
            return resolve(xv, shift - 8, p2, n2)

        return lax.cond(m == need2, exit_fn, cont_fn, (prefix2, need2))

    def process(xv):
        t_u, r, total_eq = resolve(xv, 24, jnp.int32(0), jnp.int32(_K))
        t_s = t_u ^ jnp.int32(_SIGN)
        tf = _key_to_f32(jnp.full((_L,), 0, jnp.int32) + t_s)

        def simple_pass(_o):
            @plsc.parallel_loop(0, _NCHUNK, 1, unroll=_UNROLL)
            def _body(i):
                v = xv[pl.ds(i * _L, _L)]
                xv[pl.ds(i * _L, _L)] = jnp.where(v >= tf, v, zeros16f)
            return 0

        def tie_pass(_o):
            def body(i, carry):
                v = xv[pl.ds(i * _L, _L)]
                gt = v > tf
                eq = v == tf
                pc = plsc.cumsum(eq.astype(jnp.int32)) + carry
                keep = gt | (eq & (pc <= r))
                xv[pl.ds(i * _L, _L)] = jnp.where(keep, v, zeros16f)
                return jnp.max(pc)
            lax.fori_loop(0, _NCHUNK, body, jnp.int32(0))
            return 0

        lax.cond(total_eq == r, simple_pass, tie_pass, 0)

    nrows = _ROWS // 32
    base = wid * nrows
    in_handles = [None] * nrows
    pending_out = [None] * len(bufs)

    in_handles[0] = pltpu.make_async_copy(x_hbm.at[base], bufs[0], sems[0])
    in_handles[0].start()
    for j in range(nrows):
        b = j % len(bufs)
        in_handles[j].wait()
        if j + 1 < nrows:
            nb = (j + 1) % len(bufs)
            if pending_out[nb] is not None:
                pending_out[nb].wait()
                pending_out[nb] = None
            in_handles[j + 1] = pltpu.make_async_copy(
                x_hbm.at[base + (j + 1)], bufs[nb], sems[nb])
            in_handles[j + 1].start()
        process(bufs[b])
        pending_out[b] = pltpu.make_async_copy(
            bufs[b], o_hbm.at[base + j], sems[b])
        pending_out[b].start()
    for h in pending_out:
        if h is not None:
            h.wait()


def kernel(x):
    mesh = plsc.VectorSubcoreMesh(core_axis_name="c", subcore_axis_name="s")
    f = functools.partial(
        pl.kernel,
        out_type=jax.ShapeDtypeStruct((_ROWS, _N), jnp.float32),
        mesh=mesh,
        compiler_params=pltpu.CompilerParams(needs_layout_passes=False),
        scratch_types=[
            pltpu.VMEM((_N,), jnp.float32),
            pltpu.VMEM((_N,), jnp.float32),
            pltpu.VMEM((_N,), jnp.float32),
            pltpu.VMEM((_NBINS,), jnp.int32),
            pltpu.SemaphoreType.DMA,
            pltpu.SemaphoreType.DMA,
            pltpu.SemaphoreType.DMA,
        ],
    )(_sc_body)
    return f(x)
